# Initial kernel scaffold; baseline (speedup 1.0000x reference)
import jax
import jax.numpy as jnp
from jax.experimental import pallas as pl


def kernel(queries, keys):
    raise NotImplementedError("write your pallas kernel here")


# speedup vs baseline: 1.3998x; 1.3998x over previous
import functools

import jax
import jax.numpy as jnp
from jax import lax
from jax.experimental import pallas as pl
from jax.experimental.pallas import tpu as pltpu

TOPK = 10
N = 1_000_000
Q = 64
D = 64
BLK = 8192
CHUNK = 128
CPB = BLK // CHUNK
NBLK = (N + BLK - 1) // BLK
NCHUNK = NBLK * CPB
J = 16
NEG = float("-inf")
EPS = 1e-8


def _normalize_q(q):
    return q / (jnp.sqrt(jnp.sum(q * q, axis=1, keepdims=True)) + EPS)


def _scan_body(q_ref, k_ref, ids_ref, cm_ref):
    i = pl.program_id(0)

    @pl.when(i < NBLK)
    def _scan():
        qn = _normalize_q(q_ref[...])
        k = k_ref[...]
        k2 = k * k
        ones = jnp.ones((8, D), jnp.float32)
        nsqt = lax.dot_general(ones, k2, (((1,), (1,)), ((), ())),
                               precision=lax.Precision.HIGHEST,
                               preferred_element_type=jnp.float32)
        rinvt = 1.0 / (jnp.sqrt(nsqt[0:1, :]) + EPS)
        kn = k * jnp.swapaxes(rinvt, 0, 1)
        st = lax.dot_general(qn, kn, (((1,), (1,)), ((), ())),
                             preferred_element_type=jnp.float32)
        col = lax.broadcasted_iota(jnp.int32, (1, BLK), 1) + i * BLK
        st = jnp.where(col < N, st, NEG)
        cmax = jnp.max(st.reshape(Q, CPB, CHUNK), axis=2)
        cm_ref[pl.ds(i * CPB, CPB), :] = cmax.T

    @pl.when(i >= NBLK)
    def _select():
        t = i - NBLK
        sm = cm_ref[...]
        riot = lax.broadcasted_iota(jnp.int32, (NCHUNK, Q), 0)
        m = jnp.max(sm, axis=0, keepdims=True)
        idx = jnp.min(jnp.where(sm == m, riot, NCHUNK),
                      axis=0, keepdims=True)
        ids_ref[pl.ds(t, 1), :] = idx
        cm_ref[...] = jnp.where(riot == idx, NEG, sm)


def _gather_body(ids_ref, q_ref, *refs):
    kbs = refs[:J]
    out_s_ref, out_i_ref = refs[J], refs[J + 1]
    qi = pl.program_id(0)
    qn = _normalize_q(q_ref[...])
    kcat = jnp.concatenate([kb[...] for kb in kbs], axis=0)
    k2 = kcat * kcat
    ones = jnp.ones((8, D), jnp.float32)
    nsq = lax.dot_general(ones, k2, (((1,), (1,)), ((), ())),
                          precision=lax.Precision.HIGHEST,
                          preferred_element_type=jnp.float32)
    norm = jnp.sqrt(nsq[0:1, :])
    kn = kcat / (jnp.swapaxes(norm, 0, 1) + EPS)
    st = lax.dot_general(qn, kn, (((1,), (1,)), ((), ())),
                         preferred_element_type=jnp.float32)
    qrow = lax.broadcasted_iota(jnp.int32, (Q, 1), 0)
    srow = jnp.sum(jnp.where(qrow == qi, st, 0.0),
                   axis=0, keepdims=True)
    cols = lax.broadcasted_iota(jnp.int32, (1, CHUNK), 1)
    gidx = jnp.concatenate(
        [ids_ref[qi, jj] * CHUNK + cols for jj in range(J)], axis=1)
    srow = jnp.where(gidx < N, srow, NEG)
    outs = jnp.zeros((1, 128), jnp.float32)
    outi = jnp.zeros((1, 128), jnp.int32)
    lane = lax.broadcasted_iota(jnp.int32, (1, 128), 1)
    big = jnp.int32(2**31 - 1)
    for t in range(TOPK):
        m = jnp.max(srow, axis=1, keepdims=True)
        g = jnp.min(jnp.where(srow == m, gidx, big),
                    axis=1, keepdims=True)
        outs = jnp.where(lane == t, m, outs)
        outi = jnp.where(lane == t, g, outi)
        srow = jnp.where(gidx == g, NEG, srow)
    out_s_ref[pl.ds(qi, 1), :] = outs
    out_i_ref[pl.ds(qi, 1), :] = outi


@jax.jit
def kernel(queries, keys):
    ids = pl.pallas_call(
        _scan_body,
        grid=(NBLK + J,),
        in_specs=[
            pl.BlockSpec((Q, D), lambda i: (0, 0)),
            pl.BlockSpec((BLK, D), lambda i: (jnp.minimum(i, NBLK - 1), 0)),
        ],
        out_specs=pl.BlockSpec((J, Q), lambda i: (0, 0)),
        out_shape=jax.ShapeDtypeStruct((J, Q), jnp.int32),
        scratch_shapes=[pltpu.VMEM((NCHUNK, Q), jnp.float32)],
    )(queries, keys)

    ids_qj = ids.T

    grid_spec = pltpu.PrefetchScalarGridSpec(
        num_scalar_prefetch=1,
        grid=(Q,),
        in_specs=[pl.BlockSpec((Q, D), lambda q, ids: (0, 0))] + [
            pl.BlockSpec((CHUNK, D),
                         functools.partial(
                             lambda jj, q, ids: (ids[q, jj], 0), jj))
            for jj in range(J)
        ],
        out_specs=[
            pl.BlockSpec((Q, 128), lambda q, ids: (0, 0)),
            pl.BlockSpec((Q, 128), lambda q, ids: (0, 0)),
        ],
    )
    out_s, out_i = pl.pallas_call(
        _gather_body,
        grid_spec=grid_spec,
        out_shape=[
            jax.ShapeDtypeStruct((Q, 128), jnp.float32),
            jax.ShapeDtypeStruct((Q, 128), jnp.int32),
        ],
    )(ids_qj, queries, *([keys] * J))

    return out_s[:, :TOPK], out_i[:, :TOPK]
